# baseline (speedup 1.0000x reference)
import jax
import jax.numpy as jnp
from jax.experimental import pallas as pl
from jax.experimental.pallas import tpu as pltpu

_N = 2048
_F = 256
_LEV = 2
_R = 2
_NOP = _LEV * _R - 1
_NS = _NOP * _N

_BM = 512
_ND = _NS // _BM
_NC = (_N // _BM) * _NOP
_MB = _N // _BM


def _aw_kernel(c_ref, adj_ref, xbf_ref, o_ref):
    o_ref[...] = c_ref[1] * jnp.dot(adj_ref[...].astype(jnp.bfloat16),
                                    xbf_ref[...],
                                    preferred_element_type=jnp.float32)


def _dstep_index(p):
    return jnp.where(p < 4, p, jnp.minimum((p + 3) // 2, _ND - 1))


def _main_kernel(c_ref, rv_ref, d_ref, xbf_ref, aw_ref, h0_ref, wbf_ref,
                 o_ref, dv_ref, zx_ref, acc_ref):
    p = pl.program_id(0)
    is_d = (p < 4) | ((p >= 5) & (p < 20) & (p % 2 == 1))
    is_c = jnp.logical_not(is_d) & (p >= 4)
    q = _dstep_index(p)
    pc = jnp.clip(p, 4, 23)
    j = jnp.where(pc < 12, (pc - 4) // 2,
                  jnp.where(pc < 20, 4 + (pc - 12) // 2, 8 + (pc - 20)))
    i = j // _MB
    m = j % _MB

    @pl.when(p == 0)
    def _():
        zx_ref[pl.ds(_NS, _N), :] = xbf_ref[...]

    @pl.when(is_d)
    def _():
        dv_ref[pl.ds(q * _BM, _BM), :] = d_ref[0].astype(jnp.bfloat16)

    lrow = jnp.where(is_d, q * _BM, i * _N + m * _BM)
    rrow = jnp.where(is_d, _NS, i * _N)
    res = jnp.dot(dv_ref[pl.ds(lrow, _BM), :], zx_ref[pl.ds(rrow, _N), :],
                  preferred_element_type=jnp.float32)

    @pl.when(is_d)
    def _():
        zx_ref[pl.ds(q * _BM, _BM), :] = (
            (c_ref[0] * rv_ref[...]) * res).astype(jnp.bfloat16)

    @pl.when(is_c & (i == 0))
    def _():
        acc_ref[pl.ds(m * _BM, _BM), :] = aw_ref[pl.ds(m * _BM, _BM), :] + res

    @pl.when(is_c & (i == 1))
    def _():
        acc_ref[pl.ds(m * _BM, _BM), :] += res

    @pl.when(is_c & (i == _NOP - 1))
    def _():
        s = (c_ref[3] * (acc_ref[pl.ds(m * _BM, _BM), :] + res)
             + c_ref[2] * h0_ref[...])
        o_ref[...] = (c_ref[4] * jnp.dot(s.astype(jnp.bfloat16), wbf_ref[...],
                                         preferred_element_type=jnp.float32)
                      + c_ref[5] * s)


def kernel(input, adj, d_list, h0, weight, lamda, alpha, l, gamma):
    rv2 = jax.random.uniform(jax.random.key(42), (_LEV * _R * _N, 1),
                             dtype=jnp.float32)[_N:]
    theta = jnp.log(lamda / l + 1)
    g = jnp.asarray(gamma, jnp.float32)
    a = jnp.asarray(alpha, jnp.float32)
    t = jnp.asarray(theta, jnp.float32)
    c = jnp.stack([g, 1 - g, a, 1 - a, t, 1 - t]).astype(jnp.float32)
    xbf = input.astype(jnp.bfloat16)
    wbf = weight.astype(jnp.bfloat16)

    aw = pl.pallas_call(
        _aw_kernel,
        grid=(_N // _BM,),
        in_specs=[
            pl.BlockSpec(memory_space=pltpu.SMEM),
            pl.BlockSpec((_BM, _N), lambda p: (p, 0)),
            pl.BlockSpec((_N, _F), lambda p: (0, 0)),
        ],
        out_specs=pl.BlockSpec((_BM, _F), lambda p: (p, 0)),
        out_shape=jax.ShapeDtypeStruct((_N, _F), jnp.float32),
    )(c, adj, xbf)

    out = pl.pallas_call(
        _main_kernel,
        grid=(_ND + _NC,),
        in_specs=[
            pl.BlockSpec(memory_space=pltpu.SMEM),
            pl.BlockSpec((_BM, 1), lambda p: (_dstep_index(p), 0)),
            pl.BlockSpec((1, _BM, _N),
                         lambda p: (1 + _dstep_index(p) // _MB,
                                    _dstep_index(p) % _MB, 0)),
            pl.BlockSpec((_N, _F), lambda p: (0, 0)),
            pl.BlockSpec((_N, _F), lambda p: (0, 0)),
            pl.BlockSpec((_BM, _F), lambda p: (jnp.clip(p - 20, 0, 3), 0)),
            pl.BlockSpec((_F, _F), lambda p: (0, 0)),
        ],
        out_specs=pl.BlockSpec((_BM, _F), lambda p: (jnp.clip(p - 20, 0, 3), 0)),
        out_shape=jax.ShapeDtypeStruct((_N, _F), jnp.float32),
        compiler_params=pltpu.CompilerParams(vmem_limit_bytes=67_000_000),
        scratch_shapes=[
            pltpu.VMEM((_NS, _N), jnp.bfloat16),
            pltpu.VMEM((_NS + _N, _F), jnp.bfloat16),
            pltpu.VMEM((_N, _F), jnp.float32),
        ],
    )(c, rv2, d_list, xbf, aw, h0, wbf)
    return out


# speedup vs baseline: 1.2220x; 1.2220x over previous
import jax
import jax.numpy as jnp
from jax.experimental import pallas as pl
from jax.experimental.pallas import tpu as pltpu

_N = 2048
_F = 256
_LEV = 2
_R = 2
_NOP = _LEV * _R - 1
_NS = _NOP * _N

_BM = 512
_ND = _NS // _BM
_MB = _N // _BM


def _fused_kernel(c_ref, rv_ref, d_ref, adj_ref, xbf_ref, h0_ref, wbf_ref,
                  o_ref, dv_ref, zx_ref, acc_ref):
    p = pl.program_id(0)
    is_dstep = p < _ND
    pc = jnp.clip(p - _MB, 0, _NS // _BM - 1)
    ic = pc // _MB
    mc = pc % _MB

    @pl.when(is_dstep)
    def _():
        dv_ref[pl.ds(p * _BM, _BM), :] = d_ref[0].astype(jnp.bfloat16)

    @pl.when(jnp.logical_not(is_dstep))
    def _():
        dv_ref[pl.ds(p * _BM, _BM), :] = adj_ref[...].astype(jnp.bfloat16)

    res_s = jnp.dot(dv_ref[pl.ds(p * _BM, _BM), :], xbf_ref[...],
                    preferred_element_type=jnp.float32)
    res_c = jnp.dot(dv_ref[pl.ds(ic * _N + mc * _BM, _BM), :],
                    zx_ref[pl.ds(ic * _N, _N), :],
                    preferred_element_type=jnp.float32)

    @pl.when(is_dstep)
    def _():
        zx_ref[pl.ds(p * _BM, _BM), :] = (
            (c_ref[0] * rv_ref[...]) * res_s).astype(jnp.bfloat16)

    @pl.when((p >= _MB) & (p < 2 * _MB))
    def _():
        acc_ref[pl.ds(mc * _BM, _BM), :] = res_c

    @pl.when((p >= 2 * _MB) & is_dstep)
    def _():
        acc_ref[pl.ds(mc * _BM, _BM), :] += res_c

    @pl.when(jnp.logical_not(is_dstep))
    def _():
        s = (c_ref[3] * (acc_ref[pl.ds(mc * _BM, _BM), :] + res_c
                         + c_ref[1] * res_s)
             + c_ref[2] * h0_ref[...])
        o_ref[...] = (c_ref[4] * jnp.dot(s.astype(jnp.bfloat16), wbf_ref[...],
                                         preferred_element_type=jnp.float32)
                      + c_ref[5] * s)


def kernel(input, adj, d_list, h0, weight, lamda, alpha, l, gamma):
    rv2 = jax.random.uniform(jax.random.key(42), (_LEV * _R * _N, 1),
                             dtype=jnp.float32)[_N:]
    theta = jnp.log(lamda / l + 1)
    g = jnp.asarray(gamma, jnp.float32)
    a = jnp.asarray(alpha, jnp.float32)
    t = jnp.asarray(theta, jnp.float32)
    c = jnp.stack([g, 1 - g, a, 1 - a, t, 1 - t]).astype(jnp.float32)
    xbf = input.astype(jnp.bfloat16)
    wbf = weight.astype(jnp.bfloat16)

    out = pl.pallas_call(
        _fused_kernel,
        grid=(_ND + _MB,),
        in_specs=[
            pl.BlockSpec(memory_space=pltpu.SMEM),
            pl.BlockSpec((_BM, 1), lambda p: (jnp.minimum(p, _ND - 1), 0)),
            pl.BlockSpec((1, _BM, _N),
                         lambda p: (1 + jnp.minimum(p, _ND - 1) // _MB,
                                    jnp.minimum(p, _ND - 1) % _MB, 0)),
            pl.BlockSpec((_BM, _N),
                         lambda p: (jnp.clip(p - _ND, 0, _MB - 1), 0)),
            pl.BlockSpec((_N, _F), lambda p: (0, 0)),
            pl.BlockSpec((_BM, _F), lambda p: (jnp.clip(p - _ND, 0, _MB - 1), 0)),
            pl.BlockSpec((_F, _F), lambda p: (0, 0)),
        ],
        out_specs=pl.BlockSpec((_BM, _F),
                               lambda p: (jnp.clip(p - _ND, 0, _MB - 1), 0)),
        out_shape=jax.ShapeDtypeStruct((_N, _F), jnp.float32),
        compiler_params=pltpu.CompilerParams(vmem_limit_bytes=67_000_000),
        scratch_shapes=[
            pltpu.VMEM((_NS + _N, _N), jnp.bfloat16),
            pltpu.VMEM((_NS, _F), jnp.bfloat16),
            pltpu.VMEM((_N, _F), jnp.float32),
        ],
    )(c, rv2, d_list, adj, xbf, h0, wbf)
    return out
